# Initial kernel scaffold; baseline (speedup 1.0000x reference)
import jax
import jax.numpy as jnp
from jax.experimental import pallas as pl


def kernel(h, x, edge_index, vel, W_e1, b_e1, W_e2, b_e2, W_x1, b_x1, W_x2, b_x2, W_h1, b_h1, W_h2, b_h2, ln_g, ln_b):
    raise NotImplementedError("write your pallas kernel here")


# speedup vs baseline: 3.2741x; 3.2741x over previous
import functools

import jax
import jax.numpy as jnp
from jax import lax
from jax.experimental import pallas as pl
from jax.experimental.pallas import tpu as pltpu
from jax.experimental.pallas import tpu_sc as plsc

_F32 = jnp.float32
_LANES = 16
_NW = 32
_K = 128
_ROW = 144


def _silu(v):
    return v * jax.nn.sigmoid(v)


def _stage_a(h, x, w_e1, b_e1):
    n, d = h.shape
    bn = 2000

    def body(h_ref, x_ref, w_ref, b_ref, a_ref, bb_ref):
        hb = h_ref[...]
        xpad = jnp.concatenate(
            [x_ref[...], jnp.zeros((bn, _ROW - d - 3), _F32)], axis=1)
        a = jnp.dot(hb, w_ref[0:d, :], preferred_element_type=_F32)
        bm = jnp.dot(hb, w_ref[d:2 * d, :], preferred_element_type=_F32)
        bm = bm + b_ref[...]
        a_ref[...] = jnp.concatenate([a, xpad], axis=1)
        bb_ref[...] = jnp.concatenate([bm, xpad], axis=1)

    return pl.pallas_call(
        body,
        grid=(n // bn,),
        in_specs=[
            pl.BlockSpec((bn, d), lambda i: (i, 0)),
            pl.BlockSpec((bn, 3), lambda i: (i, 0)),
            pl.BlockSpec(w_e1.shape, lambda i: (0, 0)),
            pl.BlockSpec((1, d), lambda i: (0, 0)),
        ],
        out_specs=[
            pl.BlockSpec((bn, _ROW), lambda i: (i, 0)),
            pl.BlockSpec((bn, _ROW), lambda i: (i, 0)),
        ],
        out_shape=[
            jax.ShapeDtypeStruct((n, _ROW), _F32),
            jax.ShapeDtypeStruct((n, _ROW), _F32),
        ],
    )(h, x, w_e1, b_e1.reshape(1, d))


def _stage_b(tab_a, tab_b, src, dst):
    e = src.shape[0]
    nchunk = e // _K
    tpt = (nchunk + _NW - 1) // _NW
    mesh = plsc.VectorSubcoreMesh(core_axis_name="c", subcore_axis_name="s")

    @functools.partial(
        pl.kernel,
        mesh=mesh,
        out_type=jax.ShapeDtypeStruct((e, _ROW), _F32),
        scratch_types=[
            pltpu.VMEM((_K,), jnp.int32),
            pltpu.VMEM((_K,), jnp.int32),
            pltpu.VMEM((_K, _ROW), _F32),
            pltpu.VMEM((_K, _ROW), _F32),
            pltpu.SemaphoreType.DMA,
        ],
        compiler_params=pltpu.CompilerParams(use_tc_tiling_on_sc=False),
    )
    def sc_gather(a_hbm, b_hbm, src_hbm, dst_hbm, out_hbm,
                  si, di, a_buf, b_buf, sem):
        wid = lax.axis_index("s") * 2 + lax.axis_index("c")

        @pl.loop(0, tpt)
        def _chunks(t):
            cid = t * _NW + wid

            @pl.when(cid < nchunk)
            def _():
                base = cid * _K
                pltpu.sync_copy(src_hbm.at[pl.ds(base, _K)], si)
                pltpu.sync_copy(dst_hbm.at[pl.ds(base, _K)], di)
                ca = pltpu.async_copy(a_hbm.at[si], a_buf, sem)
                cb = pltpu.async_copy(b_hbm.at[di], b_buf, sem)
                ca.wait()
                cb.wait()

                @pl.loop(0, _K)
                def _rows(r):
                    for j in range(_ROW // _LANES - 1):
                        sl = pl.ds(j * _LANES, _LANES)
                        a_buf[r, sl] = a_buf[r, sl] + b_buf[r, sl]
                    sl = pl.ds(_ROW - _LANES, _LANES)
                    a_buf[r, sl] = a_buf[r, sl] - b_buf[r, sl]

                pltpu.sync_copy(a_buf, out_hbm.at[pl.ds(base, _K)])

    return sc_gather(tab_a, tab_b, src, dst)


def _stage_c(sd, w_d, w_e2, b_e2, w_x1, b_x1, w_x2, b_x2):
    e, d = sd.shape[0], 128
    be = 3200

    def body(sd_ref, wd_ref, we2_ref, be2_ref, wx1_ref, bx1_ref,
             wx2_ref, bx2_ref, mv_ref):
        sdb = sd_ref[...]
        s = sdb[:, :d]
        rel = sdb[:, d:d + 3]
        dist2 = jnp.sum(rel * rel, axis=1, keepdims=True)
        u = _silu(s + dist2 * wd_ref[...])
        m = _silu(jnp.dot(u, we2_ref[...], preferred_element_type=_F32)
                  + be2_ref[...])
        c1 = _silu(jnp.dot(m, wx1_ref[...], preferred_element_type=_F32)
                   + bx1_ref[...])
        cw = (jnp.dot(c1, wx2_ref[...], preferred_element_type=_F32)[:, 0:1]
              + bx2_ref[...])
        vals = rel * cw
        mv_ref[...] = jnp.concatenate(
            [m, vals, jnp.ones((be, 1), _F32),
             jnp.zeros((be, _ROW - d - 4), _F32)], axis=1)

    return pl.pallas_call(
        body,
        grid=(e // be,),
        in_specs=[
            pl.BlockSpec((be, _ROW), lambda i: (i, 0)),
            pl.BlockSpec((1, d), lambda i: (0, 0)),
            pl.BlockSpec((d, d), lambda i: (0, 0)),
            pl.BlockSpec((1, d), lambda i: (0, 0)),
            pl.BlockSpec((d, d), lambda i: (0, 0)),
            pl.BlockSpec((1, d), lambda i: (0, 0)),
            pl.BlockSpec((d, d), lambda i: (0, 0)),
            pl.BlockSpec((1, 1), lambda i: (0, 0)),
        ],
        out_specs=pl.BlockSpec((be, _ROW), lambda i: (i, 0)),
        out_shape=jax.ShapeDtypeStruct((e, _ROW), _F32),
    )(sd, w_d, w_e2, b_e2.reshape(1, d), w_x1, b_x1.reshape(1, d),
      jnp.pad(w_x2, ((0, 0), (0, d - 1))), b_x2.reshape(1, 1))


def _stage_d(mv, dst, zeros_tab):
    e = mv.shape[0]
    n = zeros_tab.shape[0]
    nchunk = e // _K
    tpt = (nchunk + _NW - 1) // _NW
    rpt = n // 16
    mesh = plsc.VectorSubcoreMesh(core_axis_name="c", subcore_axis_name="s")

    @functools.partial(
        pl.kernel,
        mesh=mesh,
        out_type=jax.ShapeDtypeStruct((2, n, _ROW), _F32),
        scratch_types=[
            pltpu.VMEM((_K,), jnp.int32),
            pltpu.VMEM((_K, _ROW), _F32),
            pltpu.VMEM_SHARED((n, _ROW), _F32),
            pltpu.SemaphoreType.DMA,
        ],
        compiler_params=pltpu.CompilerParams(use_tc_tiling_on_sc=False),
    )
    def sc_scatter(mv_hbm, dst_hbm, z_hbm, out_hbm, di, mv_buf, tab, sem):
        c = lax.axis_index("c")
        s = lax.axis_index("s")
        wid = s * 2 + c
        rows = pl.ds(s * rpt, rpt)
        pltpu.sync_copy(z_hbm.at[rows], tab.at[rows])
        plsc.subcore_barrier()

        @pl.loop(0, tpt)
        def _chunks(t):
            cid = t * _NW + wid

            @pl.when(cid < nchunk)
            def _():
                base = cid * _K
                pltpu.sync_copy(dst_hbm.at[pl.ds(base, _K)], di)
                pltpu.sync_copy(mv_hbm.at[pl.ds(base, _K)], mv_buf)
                pltpu.sync_copy(mv_buf, tab.at[di], add=True)

        plsc.subcore_barrier()
        pltpu.sync_copy(tab.at[rows], out_hbm.at[c, rows])

    return sc_scatter(mv, dst, zeros_tab)


def _stage_e(parts, h, x, w_h1, b_h1, w_h2, b_h2, ln_g, ln_b):
    n, d = h.shape
    bn = 2000

    def body(p_ref, h_ref, x_ref, wh1_ref, bh1_ref, wh2_ref, bh2_ref,
             g_ref, lb_ref, ho_ref, xo_ref):
        p = p_ref[...]
        ps = p[0] + p[1]
        agg = ps[:, :d]
        sums = ps[:, d:d + 3]
        cnt = ps[:, d + 3:d + 4]
        hb = h_ref[...]
        u = _silu(jnp.dot(hb, wh1_ref[0:d, :], preferred_element_type=_F32)
                  + jnp.dot(agg, wh1_ref[d:2 * d, :],
                            preferred_element_type=_F32)
                  + bh1_ref[...])
        upd = jnp.dot(u, wh2_ref[...], preferred_element_type=_F32) + bh2_ref[...]
        h2 = hb + upd
        mu = jnp.mean(h2, axis=1, keepdims=True)
        var = jnp.mean((h2 - mu) ** 2, axis=1, keepdims=True)
        ho_ref[...] = (h2 - mu) / jnp.sqrt(var + 1e-5) * g_ref[...] + lb_ref[...]
        xo_ref[...] = x_ref[...] + sums / jnp.maximum(cnt, 1.0)

    return pl.pallas_call(
        body,
        grid=(n // bn,),
        in_specs=[
            pl.BlockSpec((2, bn, _ROW), lambda i: (0, i, 0)),
            pl.BlockSpec((bn, d), lambda i: (i, 0)),
            pl.BlockSpec((bn, 3), lambda i: (i, 0)),
            pl.BlockSpec((2 * d, d), lambda i: (0, 0)),
            pl.BlockSpec((1, d), lambda i: (0, 0)),
            pl.BlockSpec((d, d), lambda i: (0, 0)),
            pl.BlockSpec((1, d), lambda i: (0, 0)),
            pl.BlockSpec((1, d), lambda i: (0, 0)),
            pl.BlockSpec((1, d), lambda i: (0, 0)),
        ],
        out_specs=[
            pl.BlockSpec((bn, d), lambda i: (i, 0)),
            pl.BlockSpec((bn, 3), lambda i: (i, 0)),
        ],
        out_shape=[
            jax.ShapeDtypeStruct((n, d), _F32),
            jax.ShapeDtypeStruct((n, 3), _F32),
        ],
    )(parts, h, x, w_h1, b_h1.reshape(1, d), w_h2, b_h2.reshape(1, d),
      ln_g.reshape(1, d), ln_b.reshape(1, d))


def kernel(h, x, edge_index, vel, W_e1, b_e1, W_e2, b_e2, W_x1, b_x1,
           W_x2, b_x2, W_h1, b_h1, W_h2, b_h2, ln_g, ln_b):
    del vel
    n, d = h.shape
    src = edge_index[0].astype(jnp.int32)
    dst = edge_index[1].astype(jnp.int32)

    tab_a, tab_b = _stage_a(h, x, W_e1, b_e1)
    sd = _stage_b(tab_a, tab_b, src, dst)
    mv = _stage_c(sd, W_e1[2 * d:2 * d + 1, :], W_e2, b_e2,
                  W_x1, b_x1, W_x2, b_x2)
    parts = _stage_d(mv, dst, jnp.zeros((n, _ROW), _F32))
    h_out, x_new = _stage_e(parts, h, x, W_h1, b_h1, W_h2, b_h2, ln_g, ln_b)
    return (h_out, x_new)


# speedup vs baseline: 3.8027x; 1.1615x over previous
import functools

import jax
import jax.numpy as jnp
from jax import lax
from jax.experimental import pallas as pl
from jax.experimental.pallas import tpu as pltpu
from jax.experimental.pallas import tpu_sc as plsc

_F32 = jnp.float32
_L = 16
_NW = 32
_K = 128
_D = 128
_SC_PARAMS = pltpu.CompilerParams(use_tc_tiling_on_sc=False,
                                  needs_layout_passes=False)


def _silu(v):
    return v * jax.nn.sigmoid(v)


def _stage_a(h, w_e1, b_e1):
    n, d = h.shape
    bn = 2000

    def body(h_ref, w_ref, b_ref, a_ref, bb_ref):
        hb = h_ref[...]
        a_ref[...] = jnp.dot(hb, w_ref[0:d, :], preferred_element_type=_F32)
        bb_ref[...] = (jnp.dot(hb, w_ref[d:2 * d, :],
                               preferred_element_type=_F32) + b_ref[...])

    return pl.pallas_call(
        body,
        grid=(n // bn,),
        in_specs=[
            pl.BlockSpec((bn, d), lambda i: (i, 0)),
            pl.BlockSpec(w_e1.shape, lambda i: (0, 0)),
            pl.BlockSpec((1, d), lambda i: (0, 0)),
        ],
        out_specs=[
            pl.BlockSpec((bn, d), lambda i: (i, 0)),
            pl.BlockSpec((bn, d), lambda i: (i, 0)),
        ],
        out_shape=[
            jax.ShapeDtypeStruct((n, d), _F32),
            jax.ShapeDtypeStruct((n, d), _F32),
        ],
    )(h, w_e1, b_e1.reshape(1, d))


def _stage_b(tab_a, tab_b, xp, wd, src, dst):
    e = src.shape[0]
    nchunk = e // _K
    tpt = (nchunk + _NW - 1) // _NW
    if tpt % 2:
        tpt += 1
    mesh = plsc.VectorSubcoreMesh(core_axis_name="c", subcore_axis_name="s")

    @functools.partial(
        pl.kernel,
        mesh=mesh,
        out_type=jax.ShapeDtypeStruct((e, _D), _F32),
        scratch_types=[
            pltpu.VMEM((_K,), jnp.int32), pltpu.VMEM((_K,), jnp.int32),
            pltpu.VMEM((_K,), jnp.int32), pltpu.VMEM((_K,), jnp.int32),
            pltpu.VMEM((_K, _D), _F32), pltpu.VMEM((_K, _D), _F32),
            pltpu.VMEM((_K, _D), _F32), pltpu.VMEM((_K, _D), _F32),
            pltpu.VMEM((_K, _L), _F32), pltpu.VMEM((_K, _L), _F32),
            pltpu.VMEM((_K, _L), _F32), pltpu.VMEM((_K, _L), _F32),
            pltpu.VMEM((_D,), _F32),
            pltpu.SemaphoreType.DMA, pltpu.SemaphoreType.DMA,
            pltpu.SemaphoreType.DMA, pltpu.SemaphoreType.DMA,
        ],
        compiler_params=_SC_PARAMS,
    )
    def sc_gather(a_hbm, b_hbm, xp_hbm, wd_hbm, src_hbm, dst_hbm, t1_hbm,
                  si0, si1, di0, di1, a0, a1, b0, b1, xs0, xs1, xd0, xd1,
                  wdv, sg0, sg1, so0, so1):
        si = (si0, si1)
        di = (di0, di1)
        ab = (a0, a1)
        bb = (b0, b1)
        xsb = (xs0, xs1)
        xdb = (xd0, xd1)
        sg = (sg0, sg1)
        so = (so0, so1)
        wid = lax.axis_index("s") * 2 + lax.axis_index("c")
        npt = (nchunk - wid + _NW - 1) // _NW

        pltpu.sync_copy(wd_hbm, wdv)
        wd_vals = [wdv[pl.ds(j * _L, _L)] for j in range(_D // _L)]

        def issue(cid, s):
            base = cid * _K
            pltpu.sync_copy(src_hbm.at[pl.ds(base, _K)], si[s])
            pltpu.sync_copy(dst_hbm.at[pl.ds(base, _K)], di[s])
            pltpu.async_copy(a_hbm.at[si[s]], ab[s], sg[s])
            pltpu.async_copy(b_hbm.at[di[s]], bb[s], sg[s])
            pltpu.async_copy(xp_hbm.at[si[s]], xsb[s], sg[s])
            pltpu.async_copy(xp_hbm.at[di[s]], xdb[s], sg[s])

        def drain_gathers(s):
            pltpu.make_async_copy(a_hbm.at[si[s]], ab[s], sg[s]).wait()
            pltpu.make_async_copy(b_hbm.at[di[s]], bb[s], sg[s]).wait()
            pltpu.make_async_copy(xp_hbm.at[si[s]], xsb[s], sg[s]).wait()
            pltpu.make_async_copy(xp_hbm.at[di[s]], xdb[s], sg[s]).wait()

        def out_start(cid, s):
            pltpu.async_copy(ab[s], t1_hbm.at[pl.ds(cid * _K, _K)], so[s])

        def out_drain(s):
            pltpu.make_async_copy(ab[s], t1_hbm.at[pl.ds(0, _K)], so[s]).wait()

        def compute(s):
            ar, br, xsr, xdr = ab[s], bb[s], xsb[s], xdb[s]

            @pl.loop(0, _K)
            def _rows(r):
                dvec = xsr[r, pl.ds(0, _L)] - xdr[r, pl.ds(0, _L)]
                dist2 = jnp.sum(dvec * dvec)
                for j in range(_D // _L):
                    sl = pl.ds(j * _L, _L)
                    ar[r, sl] = ar[r, sl] + br[r, sl] + dist2 * wd_vals[j]

        issue(wid, 0)

        @pl.loop(0, tpt, step=2)
        def _steps(t):
            for b in range(2):
                cl = t + b
                cid = cl * _NW + wid

                @pl.when(cid < nchunk)
                def _(cl=cl, cid=cid, s=b):
                    drain_gathers(s)
                    nxt = cid + _NW

                    @pl.when(nxt < nchunk)
                    def _():
                        @pl.when(cl >= 1)
                        def _():
                            out_drain(1 - s)
                        issue(nxt, 1 - s)

                    compute(s)
                    out_start(cid, s)

        for s in range(2):
            @pl.when(npt >= s + 1)
            def _(s=s):
                out_drain(s)

    return sc_gather(tab_a, tab_b, xp, wd, src, dst)


def _stage_c(t1, w_e2, b_e2, w_x1, b_x1, w_x2, b_x2):
    e, d = t1.shape
    be = 3200

    def body(t1_ref, we2_ref, be2_ref, wx1_ref, bx1_ref, wx2_ref, bx2_ref,
             m_ref, cw_ref):
        u = _silu(t1_ref[...])
        m = _silu(jnp.dot(u, we2_ref[...], preferred_element_type=_F32)
                  + be2_ref[...])
        m_ref[...] = m
        c1 = _silu(jnp.dot(m, wx1_ref[...], preferred_element_type=_F32)
                   + bx1_ref[...])
        cw = jnp.dot(c1, wx2_ref[...], preferred_element_type=_F32)
        cw_ref[...] = cw[:, 0:1] + bx2_ref[...]

    return pl.pallas_call(
        body,
        grid=(e // be,),
        in_specs=[
            pl.BlockSpec((be, d), lambda i: (i, 0)),
            pl.BlockSpec((d, d), lambda i: (0, 0)),
            pl.BlockSpec((1, d), lambda i: (0, 0)),
            pl.BlockSpec((d, d), lambda i: (0, 0)),
            pl.BlockSpec((1, d), lambda i: (0, 0)),
            pl.BlockSpec((d, d), lambda i: (0, 0)),
            pl.BlockSpec((1, 1), lambda i: (0, 0)),
        ],
        out_specs=[
            pl.BlockSpec((be, d), lambda i: (i, 0)),
            pl.BlockSpec((be, 1), lambda i: (i, 0)),
        ],
        out_shape=[
            jax.ShapeDtypeStruct((e, d), _F32),
            jax.ShapeDtypeStruct((e, 1), _F32),
        ],
    )(t1, w_e2, b_e2.reshape(1, d), w_x1, b_x1.reshape(1, d),
      jnp.pad(w_x2, ((0, 0), (0, d - 1))), b_x2.reshape(1, 1))


def _stage_d(m, cw, src, dst, xp, zm, zv, l4):
    e = m.shape[0]
    n = zm.shape[0]
    nchunk = e // _K
    tpt = (nchunk + _NW - 1) // _NW
    rpt = n // 16
    mesh = plsc.VectorSubcoreMesh(core_axis_name="c", subcore_axis_name="s")

    @functools.partial(
        pl.kernel,
        mesh=mesh,
        out_type=[
            jax.ShapeDtypeStruct((2, n, _D), _F32),
            jax.ShapeDtypeStruct((2, n, _L), _F32),
        ],
        scratch_types=[
            pltpu.VMEM((_K,), jnp.int32), pltpu.VMEM((_K,), jnp.int32),
            pltpu.VMEM((_K,), _F32),
            pltpu.VMEM((_K, _D), _F32), pltpu.VMEM((_K, _L), _F32),
            pltpu.VMEM((_L,), _F32),
            pltpu.VMEM_SHARED((n, _D), _F32),
            pltpu.VMEM_SHARED((n, _L), _F32),
            pltpu.SemaphoreType.DMA,
        ],
        compiler_params=_SC_PARAMS,
    )
    def sc_scatter(m_hbm, cw_hbm, src_hbm, dst_hbm, xp_hbm, zm_hbm, zv_hbm,
                   l4_hbm, outm, outv, di, sj, cwv, m_buf, v_buf, l4v,
                   tabm, tabv, sem):
        c = lax.axis_index("c")
        s = lax.axis_index("s")
        wid = s * 2 + c
        rows = pl.ds(s * rpt, rpt)
        pltpu.sync_copy(zm_hbm.at[rows], tabm.at[rows])
        pltpu.sync_copy(zv_hbm.at[rows], tabv.at[rows])
        pltpu.sync_copy(l4_hbm, l4v)
        plsc.subcore_barrier()

        @pl.loop(0, tpt)
        def _chunks(t):
            cid = t * _NW + wid

            @pl.when(cid < nchunk)
            def _():
                base = cid * _K
                pltpu.sync_copy(dst_hbm.at[pl.ds(base, _K)], di)
                pltpu.sync_copy(src_hbm.at[pl.ds(base, _K)], sj)
                pltpu.sync_copy(cw_hbm.at[pl.ds(base, _K)], cwv)
                ca = pltpu.async_copy(m_hbm.at[pl.ds(base, _K)], m_buf, sem)
                cb = pltpu.async_copy(xp_hbm.at[sj], v_buf, sem)
                ca.wait()
                cb.wait()

                @pl.loop(0, _K)
                def _rows(r):
                    sl = pl.ds(0, _L)
                    cwb = plsc.load_gather(cwv, [jnp.full((_L,), r, jnp.int32)])
                    v_buf[r, sl] = cwb * v_buf[r, sl] + l4v[sl]

                pltpu.sync_copy(m_buf, tabm.at[di], add=True)
                pltpu.sync_copy(v_buf, tabv.at[di], add=True)

        plsc.subcore_barrier()
        pltpu.sync_copy(tabm.at[rows], outm.at[c, rows])
        pltpu.sync_copy(tabv.at[rows], outv.at[c, rows])

    return sc_scatter(m, cw, src, dst, xp, zm, zv, l4)


def _stage_e(pm, pv, h, x, w_h1, b_h1, w_h2, b_h2, ln_g, ln_b):
    n, d = h.shape
    bn = 2000

    def body(pm_ref, pv_ref, h_ref, x_ref, wh1_ref, bh1_ref, wh2_ref,
             bh2_ref, g_ref, lb_ref, ho_ref, xo_ref):
        pmb = pm_ref[...]
        pvb = pv_ref[...]
        agg = pmb[0] + pmb[1]
        aux = pvb[0] + pvb[1]
        sx = aux[:, 0:3]
        s1 = aux[:, 3:4]
        cnt = aux[:, 4:5]
        hb = h_ref[...]
        xb = x_ref[...]
        u = _silu(jnp.dot(hb, wh1_ref[0:d, :], preferred_element_type=_F32)
                  + jnp.dot(agg, wh1_ref[d:2 * d, :],
                            preferred_element_type=_F32)
                  + bh1_ref[...])
        upd = jnp.dot(u, wh2_ref[...], preferred_element_type=_F32) + bh2_ref[...]
        h2 = hb + upd
        mu = jnp.mean(h2, axis=1, keepdims=True)
        var = jnp.mean((h2 - mu) ** 2, axis=1, keepdims=True)
        ho_ref[...] = (h2 - mu) / jnp.sqrt(var + 1e-5) * g_ref[...] + lb_ref[...]
        xo_ref[...] = xb + (sx - xb * s1) / jnp.maximum(cnt, 1.0)

    return pl.pallas_call(
        body,
        grid=(n // bn,),
        in_specs=[
            pl.BlockSpec((2, bn, d), lambda i: (0, i, 0)),
            pl.BlockSpec((2, bn, _L), lambda i: (0, i, 0)),
            pl.BlockSpec((bn, d), lambda i: (i, 0)),
            pl.BlockSpec((bn, 3), lambda i: (i, 0)),
            pl.BlockSpec((2 * d, d), lambda i: (0, 0)),
            pl.BlockSpec((1, d), lambda i: (0, 0)),
            pl.BlockSpec((d, d), lambda i: (0, 0)),
            pl.BlockSpec((1, d), lambda i: (0, 0)),
            pl.BlockSpec((1, d), lambda i: (0, 0)),
            pl.BlockSpec((1, d), lambda i: (0, 0)),
        ],
        out_specs=[
            pl.BlockSpec((bn, d), lambda i: (i, 0)),
            pl.BlockSpec((bn, 3), lambda i: (i, 0)),
        ],
        out_shape=[
            jax.ShapeDtypeStruct((n, d), _F32),
            jax.ShapeDtypeStruct((n, 3), _F32),
        ],
    )(pm, pv, h, x, w_h1, b_h1.reshape(1, d), w_h2, b_h2.reshape(1, d),
      ln_g.reshape(1, d), ln_b.reshape(1, d))


def kernel(h, x, edge_index, vel, W_e1, b_e1, W_e2, b_e2, W_x1, b_x1,
           W_x2, b_x2, W_h1, b_h1, W_h2, b_h2, ln_g, ln_b):
    del vel
    n, d = h.shape
    src = edge_index[0].astype(jnp.int32)
    dst = edge_index[1].astype(jnp.int32)
    xp = jnp.concatenate(
        [x, jnp.ones((n, 1), _F32), jnp.zeros((n, _L - 4), _F32)], axis=1)

    tab_a, tab_b = _stage_a(h, W_e1, b_e1)
    t1 = _stage_b(tab_a, tab_b, xp, W_e1[2 * d], src, dst)
    m, cw = _stage_c(t1, W_e2, b_e2, W_x1, b_x1, W_x2, b_x2)
    lane4 = (jnp.arange(_L) == 4).astype(_F32)
    pm, pv = _stage_d(m, cw.reshape(src.shape[0]), src, dst, xp,
                      jnp.zeros((n, _D), _F32), jnp.zeros((n, _L), _F32),
                      lane4)
    return _stage_e(pm, pv, h, x, W_h1, b_h1, W_h2, b_h2, ln_g, ln_b)


# speedup vs baseline: 4.0373x; 1.0617x over previous
import functools

import jax
import jax.numpy as jnp
from jax import lax
from jax.experimental import pallas as pl
from jax.experimental.pallas import tpu as pltpu
from jax.experimental.pallas import tpu_sc as plsc

_F32 = jnp.float32
_L = 16
_NW = 32
_K = 128
_D = 128
_SC_PARAMS = pltpu.CompilerParams(use_tc_tiling_on_sc=False,
                                  needs_layout_passes=False)


def _silu(v):
    return v * jax.nn.sigmoid(v)


def _stage_a(h, w_e1, b_e1):
    n, d = h.shape
    bn = 2000

    def body(h_ref, w_ref, b_ref, a_ref, bb_ref):
        hb = h_ref[...]
        a_ref[...] = jnp.dot(hb, w_ref[0:d, :], preferred_element_type=_F32)
        bb_ref[...] = (jnp.dot(hb, w_ref[d:2 * d, :],
                               preferred_element_type=_F32) + b_ref[...])

    return pl.pallas_call(
        body,
        grid=(n // bn,),
        in_specs=[
            pl.BlockSpec((bn, d), lambda i: (i, 0)),
            pl.BlockSpec(w_e1.shape, lambda i: (0, 0)),
            pl.BlockSpec((1, d), lambda i: (0, 0)),
        ],
        out_specs=[
            pl.BlockSpec((bn, d), lambda i: (i, 0)),
            pl.BlockSpec((bn, d), lambda i: (i, 0)),
        ],
        out_shape=[
            jax.ShapeDtypeStruct((n, d), _F32),
            jax.ShapeDtypeStruct((n, d), _F32),
        ],
    )(h, w_e1, b_e1.reshape(1, d))


def _stage_b(tab_a, tab_b, xp, src, dst):
    e = src.shape[0]
    nchunk = e // _K
    tpt = (nchunk + _NW - 1) // _NW
    if tpt % 2:
        tpt += 1
    mesh = plsc.VectorSubcoreMesh(core_axis_name="c", subcore_axis_name="s")

    @functools.partial(
        pl.kernel,
        mesh=mesh,
        out_type=[
            jax.ShapeDtypeStruct((e, _D), _F32),
            jax.ShapeDtypeStruct((e, _L), _F32),
        ],
        scratch_types=[
            pltpu.VMEM((_K,), jnp.int32), pltpu.VMEM((_K,), jnp.int32),
            pltpu.VMEM((_K,), jnp.int32), pltpu.VMEM((_K,), jnp.int32),
            pltpu.VMEM((_K, _D), _F32), pltpu.VMEM((_K, _D), _F32),
            pltpu.VMEM((_K, _D), _F32), pltpu.VMEM((_K, _D), _F32),
            pltpu.VMEM((_K, _L), _F32), pltpu.VMEM((_K, _L), _F32),
            pltpu.VMEM((_K, _L), _F32), pltpu.VMEM((_K, _L), _F32),
            pltpu.SemaphoreType.DMA, pltpu.SemaphoreType.DMA,
            pltpu.SemaphoreType.DMA, pltpu.SemaphoreType.DMA,
        ],
        compiler_params=_SC_PARAMS,
    )
    def sc_gather(a_hbm, b_hbm, xp_hbm, src_hbm, dst_hbm, s_hbm, d_hbm,
                  si0, si1, di0, di1, a0, a1, b0, b1, xs0, xs1, xd0, xd1,
                  sg0, sg1, so0, so1):
        si = (si0, si1)
        di = (di0, di1)
        ab = (a0, a1)
        bb = (b0, b1)
        xsb = (xs0, xs1)
        xdb = (xd0, xd1)
        sg = (sg0, sg1)
        so = (so0, so1)
        wid = lax.axis_index("s") * 2 + lax.axis_index("c")
        npt = (nchunk - wid + _NW - 1) // _NW

        def issue(cid, s):
            base = cid * _K
            pltpu.sync_copy(src_hbm.at[pl.ds(base, _K)], si[s])
            pltpu.sync_copy(dst_hbm.at[pl.ds(base, _K)], di[s])
            pltpu.async_copy(a_hbm.at[si[s]], ab[s], sg[s])
            pltpu.async_copy(b_hbm.at[di[s]], bb[s], sg[s])
            pltpu.async_copy(xp_hbm.at[si[s]], xsb[s], sg[s])
            pltpu.async_copy(xp_hbm.at[di[s]], xdb[s], sg[s])

        def drain_gathers(s):
            pltpu.make_async_copy(a_hbm.at[si[s]], ab[s], sg[s]).wait()
            pltpu.make_async_copy(b_hbm.at[di[s]], bb[s], sg[s]).wait()
            pltpu.make_async_copy(xp_hbm.at[si[s]], xsb[s], sg[s]).wait()
            pltpu.make_async_copy(xp_hbm.at[di[s]], xdb[s], sg[s]).wait()

        def out_start(cid, s):
            pltpu.async_copy(ab[s], s_hbm.at[pl.ds(cid * _K, _K)], so[s])
            pltpu.async_copy(xsb[s], d_hbm.at[pl.ds(cid * _K, _K)], so[s])

        def out_drain(s):
            pltpu.make_async_copy(ab[s], s_hbm.at[pl.ds(0, _K)], so[s]).wait()
            pltpu.make_async_copy(xsb[s], d_hbm.at[pl.ds(0, _K)], so[s]).wait()

        def compute(s):
            ar, br, xsr, xdr = ab[s], bb[s], xsb[s], xdb[s]

            @pl.loop(0, _K)
            def _rows(r):
                for j in range(_D // _L):
                    sl = pl.ds(j * _L, _L)
                    ar[r, sl] = ar[r, sl] + br[r, sl]
                sl = pl.ds(0, _L)
                xsr[r, sl] = xsr[r, sl] - xdr[r, sl]

        issue(wid, 0)

        @pl.loop(0, tpt, step=2)
        def _steps(t):
            for b in range(2):
                cl = t + b
                cid = cl * _NW + wid

                @pl.when(cid < nchunk)
                def _(cl=cl, cid=cid, s=b):
                    drain_gathers(s)
                    nxt = cid + _NW

                    @pl.when(nxt < nchunk)
                    def _():
                        @pl.when(cl >= 1)
                        def _():
                            out_drain(1 - s)
                        issue(nxt, 1 - s)

                    compute(s)
                    out_start(cid, s)

        for s in range(2):
            @pl.when(npt >= s + 1)
            def _(s=s):
                out_drain(s)

    return sc_gather(tab_a, tab_b, xp, src, dst)


def _stage_c(s_arr, x8, rbig, w_e2, b_e2, w_x1, b_x1, w_x2, b_x2):
    e, d = s_arr.shape
    be = 3200

    def body(s_ref, x8_ref, rb_ref, we2_ref, be2_ref, wx1_ref, bx1_ref,
             wx2_ref, bx2_ref, m_ref, cw_ref):
        x8b = x8_ref[...]
        z = jnp.dot(x8b * x8b, rb_ref[...], preferred_element_type=_F32)
        t1 = s_ref[...] + z.reshape(be, d)
        u = _silu(t1)
        m = _silu(jnp.dot(u, we2_ref[...], preferred_element_type=_F32)
                  + be2_ref[...])
        m_ref[...] = m
        c1 = _silu(jnp.dot(m, wx1_ref[...], preferred_element_type=_F32)
                   + bx1_ref[...])
        cw = (jnp.dot(c1, wx2_ref[...], preferred_element_type=_F32)[:, 0:1]
              + bx2_ref[...])
        cw_ref[...] = jnp.concatenate(
            [cw, cw, cw, cw, jnp.ones((be, 1), _F32),
             jnp.zeros((be, _L - 5), _F32)], axis=1)

    return pl.pallas_call(
        body,
        grid=(e // be,),
        in_specs=[
            pl.BlockSpec((be, d), lambda i: (i, 0)),
            pl.BlockSpec((be // 8, d), lambda i: (i, 0)),
            pl.BlockSpec((d, 8 * d), lambda i: (0, 0)),
            pl.BlockSpec((d, d), lambda i: (0, 0)),
            pl.BlockSpec((1, d), lambda i: (0, 0)),
            pl.BlockSpec((d, d), lambda i: (0, 0)),
            pl.BlockSpec((1, d), lambda i: (0, 0)),
            pl.BlockSpec((d, d), lambda i: (0, 0)),
            pl.BlockSpec((1, 1), lambda i: (0, 0)),
        ],
        out_specs=[
            pl.BlockSpec((be, d), lambda i: (i, 0)),
            pl.BlockSpec((be, _L), lambda i: (i, 0)),
        ],
        out_shape=[
            jax.ShapeDtypeStruct((e, d), _F32),
            jax.ShapeDtypeStruct((e, _L), _F32),
        ],
    )(s_arr, x8, rbig, w_e2, b_e2.reshape(1, d), w_x1, b_x1.reshape(1, d),
      jnp.pad(w_x2, ((0, 0), (0, d - 1))), b_x2.reshape(1, 1))


def _stage_d1(m, dst, zm):
    e = m.shape[0]
    n = zm.shape[0]
    nchunk = e // _K
    tpt = (nchunk + _NW - 1) // _NW
    if tpt % 2:
        tpt += 1
    rpt = n // 16
    mesh = plsc.VectorSubcoreMesh(core_axis_name="c", subcore_axis_name="s")

    @functools.partial(
        pl.kernel,
        mesh=mesh,
        out_type=jax.ShapeDtypeStruct((2, n, _D), _F32),
        scratch_types=[
            pltpu.VMEM((_K,), jnp.int32), pltpu.VMEM((_K,), jnp.int32),
            pltpu.VMEM((_K, _D), _F32), pltpu.VMEM((_K, _D), _F32),
            pltpu.VMEM_SHARED((n, _D), _F32),
            pltpu.SemaphoreType.DMA, pltpu.SemaphoreType.DMA,
        ],
        compiler_params=_SC_PARAMS,
    )
    def sc_scatter_m(m_hbm, dst_hbm, zm_hbm, outm,
                     di0, di1, m0, m1, tabm, sg0, sg1):
        di = (di0, di1)
        mb = (m0, m1)
        sg = (sg0, sg1)
        c = lax.axis_index("c")
        s = lax.axis_index("s")
        wid = s * 2 + c
        rows = pl.ds(s * rpt, rpt)
        pltpu.sync_copy(zm_hbm.at[rows], tabm.at[rows])
        plsc.subcore_barrier()

        def issue(cid, s2):
            base = cid * _K
            pltpu.sync_copy(dst_hbm.at[pl.ds(base, _K)], di[s2])
            pltpu.async_copy(m_hbm.at[pl.ds(base, _K)], mb[s2], sg[s2])

        def drain(s2):
            pltpu.make_async_copy(m_hbm.at[pl.ds(0, _K)], mb[s2], sg[s2]).wait()

        issue(wid, 0)

        @pl.loop(0, tpt, step=2)
        def _steps(t):
            for b in range(2):
                cid = (t + b) * _NW + wid

                @pl.when(cid < nchunk)
                def _(cid=cid, s2=b):
                    drain(s2)

                    @pl.when(cid + _NW < nchunk)
                    def _():
                        issue(cid + _NW, 1 - s2)

                    pltpu.sync_copy(mb[s2], tabm.at[di[s2]], add=True)

        plsc.subcore_barrier()
        pltpu.sync_copy(tabm.at[rows], outm.at[c, rows])

    return sc_scatter_m(m, dst, zm)


def _stage_d2(cw16, src, dst, xp, zv):
    e = cw16.shape[0]
    n = zv.shape[0]
    nchunk = e // _K
    tpt = (nchunk + _NW - 1) // _NW
    if tpt % 2:
        tpt += 1
    rpt = n // 16
    mesh = plsc.VectorSubcoreMesh(core_axis_name="c", subcore_axis_name="s")

    @functools.partial(
        pl.kernel,
        mesh=mesh,
        out_type=jax.ShapeDtypeStruct((2, n, _L), _F32),
        scratch_types=[
            pltpu.VMEM((_K,), jnp.int32), pltpu.VMEM((_K,), jnp.int32),
            pltpu.VMEM((_K,), jnp.int32), pltpu.VMEM((_K,), jnp.int32),
            pltpu.VMEM((_K, _L), _F32), pltpu.VMEM((_K, _L), _F32),
            pltpu.VMEM((_K, _L), _F32), pltpu.VMEM((_K, _L), _F32),
            pltpu.VMEM_SHARED((n, _L), _F32),
            pltpu.SemaphoreType.DMA, pltpu.SemaphoreType.DMA,
        ],
        compiler_params=_SC_PARAMS,
    )
    def sc_scatter_v(cw_hbm, src_hbm, dst_hbm, xp_hbm, zv_hbm, outv,
                     di0, di1, sj0, sj1, cw0, cw1, xs0, xs1, tabv, sg0, sg1):
        di = (di0, di1)
        sj = (sj0, sj1)
        cwb = (cw0, cw1)
        xsb = (xs0, xs1)
        sg = (sg0, sg1)
        c = lax.axis_index("c")
        s = lax.axis_index("s")
        wid = s * 2 + c
        rows = pl.ds(s * rpt, rpt)
        pltpu.sync_copy(zv_hbm.at[rows], tabv.at[rows])
        plsc.subcore_barrier()

        def issue(cid, s2):
            base = cid * _K
            pltpu.sync_copy(dst_hbm.at[pl.ds(base, _K)], di[s2])
            pltpu.sync_copy(src_hbm.at[pl.ds(base, _K)], sj[s2])
            pltpu.async_copy(cw_hbm.at[pl.ds(base, _K)], cwb[s2], sg[s2])
            pltpu.async_copy(xp_hbm.at[sj[s2]], xsb[s2], sg[s2])

        def drain(s2):
            pltpu.make_async_copy(cw_hbm.at[pl.ds(0, _K)], cwb[s2], sg[s2]).wait()
            pltpu.make_async_copy(xp_hbm.at[sj[s2]], xsb[s2], sg[s2]).wait()

        issue(wid, 0)

        @pl.loop(0, tpt, step=2)
        def _steps(t):
            for b in range(2):
                cid = (t + b) * _NW + wid

                @pl.when(cid < nchunk)
                def _(cid=cid, s2=b):
                    drain(s2)

                    @pl.when(cid + _NW < nchunk)
                    def _():
                        issue(cid + _NW, 1 - s2)

                    xsr, cwr = xsb[s2], cwb[s2]

                    @pl.loop(0, _K)
                    def _rows(r):
                        sl = pl.ds(0, _L)
                        xsr[r, sl] = cwr[r, sl] * xsr[r, sl]

                    pltpu.sync_copy(xsr, tabv.at[di[s2]], add=True)

        plsc.subcore_barrier()
        pltpu.sync_copy(tabv.at[rows], outv.at[c, rows])

    return sc_scatter_v(cw16, src, dst, xp, zv)


def _stage_e(pm, pv, h, x, w_h1, b_h1, w_h2, b_h2, ln_g, ln_b):
    n, d = h.shape
    bn = 2000

    def body(pm_ref, pv_ref, h_ref, x_ref, wh1_ref, bh1_ref, wh2_ref,
             bh2_ref, g_ref, lb_ref, ho_ref, xo_ref):
        pmb = pm_ref[...]
        pvb = pv_ref[...]
        agg = pmb[0] + pmb[1]
        aux = pvb[0] + pvb[1]
        sx = aux[:, 0:3]
        s1 = aux[:, 3:4]
        cnt = aux[:, 4:5]
        hb = h_ref[...]
        xb = x_ref[...]
        u = _silu(jnp.dot(hb, wh1_ref[0:d, :], preferred_element_type=_F32)
                  + jnp.dot(agg, wh1_ref[d:2 * d, :],
                            preferred_element_type=_F32)
                  + bh1_ref[...])
        upd = jnp.dot(u, wh2_ref[...], preferred_element_type=_F32) + bh2_ref[...]
        h2 = hb + upd
        mu = jnp.mean(h2, axis=1, keepdims=True)
        var = jnp.mean((h2 - mu) ** 2, axis=1, keepdims=True)
        ho_ref[...] = (h2 - mu) / jnp.sqrt(var + 1e-5) * g_ref[...] + lb_ref[...]
        xo_ref[...] = xb + (sx - xb * s1) / jnp.maximum(cnt, 1.0)

    return pl.pallas_call(
        body,
        grid=(n // bn,),
        in_specs=[
            pl.BlockSpec((2, bn, d), lambda i: (0, i, 0)),
            pl.BlockSpec((2, bn, _L), lambda i: (0, i, 0)),
            pl.BlockSpec((bn, d), lambda i: (i, 0)),
            pl.BlockSpec((bn, 3), lambda i: (i, 0)),
            pl.BlockSpec((2 * d, d), lambda i: (0, 0)),
            pl.BlockSpec((1, d), lambda i: (0, 0)),
            pl.BlockSpec((d, d), lambda i: (0, 0)),
            pl.BlockSpec((1, d), lambda i: (0, 0)),
            pl.BlockSpec((1, d), lambda i: (0, 0)),
            pl.BlockSpec((1, d), lambda i: (0, 0)),
        ],
        out_specs=[
            pl.BlockSpec((bn, d), lambda i: (i, 0)),
            pl.BlockSpec((bn, 3), lambda i: (i, 0)),
        ],
        out_shape=[
            jax.ShapeDtypeStruct((n, d), _F32),
            jax.ShapeDtypeStruct((n, 3), _F32),
        ],
    )(pm, pv, h, x, w_h1, b_h1.reshape(1, d), w_h2, b_h2.reshape(1, d),
      ln_g.reshape(1, d), ln_b.reshape(1, d))


def kernel(h, x, edge_index, vel, W_e1, b_e1, W_e2, b_e2, W_x1, b_x1,
           W_x2, b_x2, W_h1, b_h1, W_h2, b_h2, ln_g, ln_b):
    del vel
    n, d = h.shape
    e = edge_index.shape[1]
    src = edge_index[0].astype(jnp.int32)
    dst = edge_index[1].astype(jnp.int32)
    xp = jnp.concatenate(
        [x, jnp.ones((n, 2), _F32), jnp.zeros((n, _L - 5), _F32)], axis=1)
    wd = W_e1[2 * d]
    li = jnp.arange(8 * _L)
    ki = jnp.arange(8)
    q = ((li[:, None] // _L == ki[None, :])
         & (li[:, None] % _L < 3)).astype(_F32)
    rbig = (q[:, :, None] * wd[None, None, :]).reshape(_D, 8 * _D)

    tab_a, tab_b = _stage_a(h, W_e1, b_e1)
    s_arr, d16 = _stage_b(tab_a, tab_b, xp, src, dst)
    m, cw16 = _stage_c(s_arr, d16.reshape(e // 8, 8 * _L), rbig,
                       W_e2, b_e2, W_x1, b_x1, W_x2, b_x2)
    pm = _stage_d1(m, dst, jnp.zeros((n, _D), _F32))
    pv = _stage_d2(cw16, src, dst, xp, jnp.zeros((n, _L), _F32))
    return _stage_e(pm, pv, h, x, W_h1, b_h1, W_h2, b_h2, ln_g, ln_b)
